# baseline (speedup 1.0000x reference)
import functools

import jax
import jax.numpy as jnp
from jax import lax
from jax.experimental import pallas as pl
from jax.experimental.pallas import tpu as pltpu
from jax.experimental.pallas import tpu_sc as plsc

_EPS = 1e-12
_NC = 2
_NS = 16
_NW = _NC * _NS
_CHUNK = 80
_TCBLK = 1600


def _make_sc_gather(n_tokens, slice_tokens, slice_idx, emb):
    per_w = slice_tokens // _NW
    n_chunks = per_w // _CHUNK
    n4 = n_chunks // 4
    mesh = plsc.VectorSubcoreMesh(core_axis_name="c", subcore_axis_name="s")

    @functools.partial(
        pl.kernel,
        mesh=mesh,
        compiler_params=pltpu.CompilerParams(needs_layout_passes=False),
        out_type=jax.ShapeDtypeStruct((n_tokens, emb), jnp.float32),
        scratch_types=[
            pltpu.VMEM((n_chunks, _CHUNK), jnp.int32),
            pltpu.VMEM((_CHUNK, emb), jnp.float32),
            pltpu.VMEM((_CHUNK, emb), jnp.float32),
            pltpu.VMEM((_CHUNK, emb), jnp.float32),
            pltpu.VMEM((_CHUNK, emb), jnp.float32),
            pltpu.SemaphoreType.DMA,
            pltpu.SemaphoreType.DMA,
            pltpu.SemaphoreType.DMA,
            pltpu.SemaphoreType.DMA,
            pltpu.SemaphoreType.DMA,
            pltpu.SemaphoreType.DMA,
            pltpu.SemaphoreType.DMA,
            pltpu.SemaphoreType.DMA,
        ],
    )
    def sc_kernel(wid_hbm, word_hbm, out_hbm, ids_v,
                  row0, row1, row2, row3,
                  sw0, sw1, sw2, sw3, so0, so1, so2, so3):
        wid = lax.axis_index("s") * _NC + lax.axis_index("c")
        base = slice_idx * slice_tokens + wid * per_w
        pltpu.sync_copy(wid_hbm.at[wid], ids_v)

        rows = (row0, row1, row2, row3)
        sws = (sw0, sw1, sw2, sw3)
        sos = (so0, so1, so2, so3)

        def start_gather(ci, b):
            pltpu.make_async_copy(
                word_hbm.at[ids_v.at[ci]], rows[b], sws[b]).start()

        def wait_gather(ci, b):
            pltpu.make_async_copy(
                word_hbm.at[ids_v.at[ci]], rows[b], sws[b]).wait()

        def start_writeback(ci, b):
            pltpu.make_async_copy(
                rows[b], out_hbm.at[pl.ds(base + ci * _CHUNK, _CHUNK)],
                sos[b]).start()

        def wait_writeback(b):
            pltpu.make_async_copy(
                rows[b], out_hbm.at[pl.ds(base, _CHUNK)], sos[b]).wait()

        start_gather(0, 0)
        start_gather(1, 1)
        start_gather(2, 2)

        def loop_body(ci4, carry):
            for u in range(4):
                ci = ci4 * 4 + u
                b = u
                b3 = (u + 3) % 4
                wait_gather(ci, b)
                start_writeback(ci, b)

                @pl.when(ci + 3 < n_chunks)
                def _():
                    @pl.when(ci >= 1)
                    def _():
                        wait_writeback(b3)

                    start_gather(ci + 3, b3)
            return carry

        lax.fori_loop(0, n4, loop_body, 0)
        for b in range(4):
            wait_writeback(b)

    return sc_kernel


def _tc_ln_body(x_ref, pos_ref, ttf_ref, cst_ref, o_ref):
    x = (x_ref[...] + pos_ref[...]
         + ttf_ref[...] * cst_ref[0, :][None, :])
    mean = jnp.mean(x, axis=1, keepdims=True)
    var = jnp.mean(x * x, axis=1, keepdims=True) - mean * mean
    inv = lax.rsqrt(var + _EPS)
    o_ref[...] = ((x - mean) * inv * cst_ref[1, :][None, :]
                  + cst_ref[2, :][None, :])


def _tc_ln_body_acc(x_ref, pos_ref, ttf_ref, cst_ref, acc_ref, o_ref):
    _tc_ln_body(x_ref, pos_ref, ttf_ref, cst_ref, o_ref)


def _tc_ln(rows, posfull, ttf, cst, acc, slice_tokens, slice_idx, n_tokens,
           emb):
    grid = (slice_tokens // _TCBLK,)
    off = slice_idx * (slice_tokens // _TCBLK)
    return pl.pallas_call(
        _tc_ln_body_acc,
        grid=grid,
        in_specs=[
            pl.BlockSpec((_TCBLK, emb), lambda b: (off + b, 0)),
            pl.BlockSpec((_TCBLK, emb), lambda b: (0, 0)),
            pl.BlockSpec((_TCBLK, 1), lambda b: (off + b, 0)),
            pl.BlockSpec((3, emb), lambda b: (0, 0)),
            pl.BlockSpec((_TCBLK, emb), lambda b: (off + b, 0)),
        ],
        out_specs=pl.BlockSpec((_TCBLK, emb), lambda b: (off + b, 0)),
        out_shape=jax.ShapeDtypeStruct((n_tokens, emb), jnp.float32),
        input_output_aliases={4: 0},
    )(rows, posfull, ttf, cst, acc)


@jax.jit
def kernel(input_ids, token_type_ids, word_embeddings, position_embeddings,
           token_type_embeddings, ln_gamma, ln_beta):
    bsz, seq = input_ids.shape
    vocab, emb = word_embeddings.shape
    n_tokens = bsz * seq
    per_w = n_tokens // _NW
    n_chunks = per_w // _CHUNK

    n_slices = 4
    slice_tokens = n_tokens // n_slices
    ids = input_ids.astype(jnp.int32).reshape(
        n_slices, _NW, slice_tokens // _NW // _CHUNK, _CHUNK)
    pos2 = position_embeddings[:seq] + token_type_embeddings[0][None, :]
    posfull = jnp.tile(pos2, (_TCBLK // seq, 1))
    ttf = token_type_ids.astype(jnp.float32).reshape(n_tokens, 1)
    cst = jnp.stack(
        [token_type_embeddings[1] - token_type_embeddings[0],
         ln_gamma, ln_beta])

    interms = []
    for i in range(n_slices):
        sc = _make_sc_gather(n_tokens, slice_tokens, i, emb)
        interms.append(sc(ids[i], word_embeddings))
    acc = interms[0]
    for i in range(n_slices):
        acc = _tc_ln(interms[i], posfull, ttf, cst, acc,
                     slice_tokens, i, n_tokens, emb)
    return acc.reshape(bsz, seq, emb)


# speedup vs baseline: 1.9491x; 1.9491x over previous
import functools

import jax
import jax.numpy as jnp
from jax import lax
from jax.experimental import pallas as pl
from jax.experimental.pallas import tpu as pltpu
from jax.experimental.pallas import tpu_sc as plsc

_EPS = 1e-12
_NC = 2
_NS = 16
_NW = _NC * _NS
_LANES = 16
_CHUNK = 128
_UNROLL = 2


def _lane_shuffle(v, idx):
    dnums = lax.GatherDimensionNumbers(
        offset_dims=(), collapsed_slice_dims=(0,), start_index_map=(0,))
    return lax.gather(v, idx[:, None], dnums, slice_sizes=(1,),
                      mode=lax.GatherScatterMode.PROMISE_IN_BOUNDS)


def _allsum(v):
    lane = lax.iota(jnp.int32, _LANES)
    for stride in (1, 2, 4, 8):
        v = v + _lane_shuffle(v, lax.bitwise_xor(lane, stride))
    return v


def _rsqrt(x):
    i = plsc.bitcast(x, jnp.int32)
    i = 0x5F3759DF - lax.shift_right_arithmetic(i, 1)
    y = plsc.bitcast(i, jnp.float32)
    for _ in range(2):
        y = y * (1.5 - 0.5 * x * y * y)
    return y


def _make_sc_kernel(n_tokens, emb):
    per_w = n_tokens // _NW
    n_chunks = per_w // _CHUNK
    n2 = n_chunks // 2
    n_sub = emb // _LANES
    mesh = plsc.VectorSubcoreMesh(core_axis_name="c", subcore_axis_name="s")

    @functools.partial(
        pl.kernel,
        mesh=mesh,
        compiler_params=pltpu.CompilerParams(needs_layout_passes=False),
        out_type=jax.ShapeDtypeStruct((n_tokens, emb), jnp.float32),
        scratch_types=[
            pltpu.VMEM((n_chunks, 2, _CHUNK), jnp.int32),
            pltpu.VMEM((_CHUNK, emb), jnp.float32),
            pltpu.VMEM((_CHUNK, emb), jnp.float32),
            pltpu.VMEM((_CHUNK, emb), jnp.float32),
            pltpu.VMEM((_CHUNK, emb), jnp.float32),
            pltpu.VMEM((_CHUNK, emb), jnp.float32),
            pltpu.VMEM((_CHUNK, emb), jnp.float32),
            pltpu.VMEM((2, emb), jnp.float32),
            pltpu.SemaphoreType.DMA,
            pltpu.SemaphoreType.DMA,
            pltpu.SemaphoreType.DMA,
            pltpu.SemaphoreType.DMA,
            pltpu.SemaphoreType.DMA,
            pltpu.SemaphoreType.DMA,
        ],
    )
    def sc_kernel(pk_hbm, word_hbm, ptt_hbm, gb_hbm, out_hbm,
                  idxall, row0, row1, prw0, prw1, ob0, ob1, gb_v,
                  sw0, sw1, sp0, sp1, so0, so1):
        wid = lax.axis_index("s") * _NC + lax.axis_index("c")
        base = wid * per_w
        pltpu.sync_copy(gb_hbm, gb_v)
        pltpu.sync_copy(pk_hbm.at[wid], idxall)
        gs = [gb_v[0, pl.ds(k * _LANES, _LANES)] for k in range(n_sub)]
        bs = [gb_v[1, pl.ds(k * _LANES, _LANES)] for k in range(n_sub)]

        rows = (row0, row1)
        prws = (prw0, prw1)
        obs = (ob0, ob1)
        sws = (sw0, sw1)
        sps = (sp0, sp1)
        sos = (so0, so1)

        def start_gather(ci, b):
            pltpu.make_async_copy(
                word_hbm.at[idxall.at[ci, 0]], rows[b], sws[b]).start()
            pltpu.make_async_copy(
                ptt_hbm.at[idxall.at[ci, 1]], prws[b], sps[b]).start()

        def wait_gather(ci, b):
            pltpu.make_async_copy(
                word_hbm.at[idxall.at[ci, 0]], rows[b], sws[b]).wait()
            pltpu.make_async_copy(
                ptt_hbm.at[idxall.at[ci, 1]], prws[b], sps[b]).wait()

        def wait_writeback(b):
            pltpu.make_async_copy(
                obs[b], out_hbm.at[pl.ds(base, _CHUNK)], sos[b]).wait()

        def compute(b):
            rv, pv, ov = rows[b], prws[b], obs[b]

            def tok_body(tt, carry):
                for j in range(_UNROLL):
                    t = tt * _UNROLL + j
                    regs = [rv[t, pl.ds(k * _LANES, _LANES)]
                            + pv[t, pl.ds(k * _LANES, _LANES)]
                            for k in range(n_sub)]
                    sv = regs[0]
                    qv = regs[0] * regs[0]
                    for k in range(1, n_sub):
                        sv = sv + regs[k]
                        qv = qv + regs[k] * regs[k]
                    inv_n = 1.0 / emb
                    mean_v = _allsum(sv) * inv_n
                    msq_v = _allsum(qv) * inv_n
                    var_v = msq_v - mean_v * mean_v
                    inv_std = _rsqrt(var_v + _EPS)
                    for k in range(n_sub):
                        ov[t, pl.ds(k * _LANES, _LANES)] = (
                            (regs[k] - mean_v) * inv_std * gs[k] + bs[k])
                return carry

            lax.fori_loop(0, _CHUNK // _UNROLL, tok_body, 0)

        def start_writeback(ci, b):
            pltpu.make_async_copy(
                obs[b], out_hbm.at[pl.ds(base + ci * _CHUNK, _CHUNK)],
                sos[b]).start()

        start_gather(0, 0)

        def loop_body(ci2, carry):
            ci_a = ci2 * 2
            ci_b = ci_a + 1
            start_gather(ci_b, 1)
            wait_gather(ci_a, 0)

            @pl.when(ci2 > 0)
            def _():
                wait_writeback(0)

            compute(0)
            start_writeback(ci_a, 0)

            @pl.when(ci2 < n2 - 1)
            def _():
                start_gather(ci_a + 2, 0)

            wait_gather(ci_b, 1)

            @pl.when(ci2 > 0)
            def _():
                wait_writeback(1)

            compute(1)
            start_writeback(ci_b, 1)
            return carry

        lax.fori_loop(0, n2, loop_body, 0)
        wait_writeback(0)
        wait_writeback(1)

    return sc_kernel


@jax.jit
def kernel(input_ids, token_type_ids, word_embeddings, position_embeddings,
           token_type_embeddings, ln_gamma, ln_beta):
    bsz, seq = input_ids.shape
    vocab, emb = word_embeddings.shape
    n_tokens = bsz * seq
    per_w = n_tokens // _NW
    n_chunks = per_w // _CHUNK

    ids = input_ids.astype(jnp.int32).reshape(-1)
    tv = token_type_embeddings.shape[0]
    ptt = (position_embeddings[:seq, None, :]
           + token_type_embeddings[None, :, :]).reshape(seq * tv, emb)
    pids = (jnp.arange(seq, dtype=jnp.int32)[None, :] * tv
            + token_type_ids.astype(jnp.int32)).reshape(-1)
    packed = jnp.stack([ids.reshape(_NW, n_chunks, _CHUNK),
                        pids.reshape(_NW, n_chunks, _CHUNK)], axis=2)
    gb = jnp.stack([ln_gamma, ln_beta])

    sc = _make_sc_kernel(n_tokens, emb)
    out = sc(packed, word_embeddings, ptt, gb)
    return out.reshape(bsz, seq, emb)
